# Initial kernel scaffold; baseline (speedup 1.0000x reference)
import functools

import jax
import jax.numpy as jnp
from jax import lax
from jax.experimental import pallas as pl
from jax.experimental.pallas import tpu as pltpu
from jax.experimental.pallas import tpu_sc as plsc

N = 10000
F = 128
K = 10
ALPHA = 0.1

NC = 2
NS = 16
CH = 128
NPAD = N + 16
ROWS_PER_TILE = NPAD // NS

RB = 2000

_mesh = plsc.VectorSubcoreMesh(core_axis_name="c", subcore_axis_name="s")


def _fill_const(buf, nrows, val):
    @pl.loop(0, nrows)
    def _(i):
        @pl.loop(0, buf.shape[1], step=16)
        def _(j):
            buf[i, pl.ds(j, 16)] = jnp.full((16,), val, jnp.float32)


def _zero_spmem(acc, zbuf, tile_id):
    base = tile_id * ROWS_PER_TILE
    zrows = zbuf.shape[0]
    off = 0
    while off < ROWS_PER_TILE:
        sz = min(zrows, ROWS_PER_TILE - off)
        pltpu.sync_copy(zbuf.at[pl.ds(0, sz)], acc.at[pl.ds(base + off, sz)])
        off += sz


def _deg_call(col_p, cpt):
    @functools.partial(
        pl.kernel,
        out_type=jax.ShapeDtypeStruct((NPAD, 16), jnp.float32),
        mesh=_mesh,
        scratch_types=[
            pltpu.VMEM((cpt, CH), jnp.int32),
            pltpu.VMEM((CH, 16), jnp.float32),
            pltpu.VMEM((64, 16), jnp.float32),
            pltpu.VMEM_SHARED((NPAD, 16), jnp.float32),
        ],
    )
    def k(col_hbm, deg_hbm, cbuf, ones, zbuf, dacc):
        cid = lax.axis_index("c")
        sid = lax.axis_index("s")

        @pl.when(cid == 0)
        def _():
            _fill_const(ones, CH, 1.0)
            _fill_const(zbuf, 64, 0.0)
            base = sid * ROWS_PER_TILE
            off = 0
            while off < ROWS_PER_TILE:
                sz = min(64, ROWS_PER_TILE - off)
                pltpu.sync_copy(zbuf.at[pl.ds(0, sz)],
                                dacc.at[pl.ds(base + off, sz)])
                off += sz
            plsc.subcore_barrier()

            for seg in range(NC):
                pltpu.sync_copy(col_hbm.at[seg, sid], cbuf)

                @pl.loop(0, cpt)
                def _(j):
                    pltpu.sync_copy(ones, dacc.at[cbuf.at[j]], add=True)

            plsc.subcore_barrier()
            pltpu.sync_copy(dacc.at[pl.ds(base, ROWS_PER_TILE)],
                            deg_hbm.at[pl.ds(base, ROWS_PER_TILE)])

    return k(col_p)


def _hop_call(hs, row_p, col_p, cpt):
    @functools.partial(
        pl.kernel,
        out_type=jax.ShapeDtypeStruct((NC, NPAD, F), jnp.float32),
        mesh=_mesh,
        scratch_types=[
            pltpu.VMEM((cpt, CH), jnp.int32),
            pltpu.VMEM((cpt, CH), jnp.int32),
            pltpu.VMEM((CH, F), jnp.float32),
            pltpu.VMEM((64, F), jnp.float32),
            pltpu.VMEM_SHARED((NPAD, F), jnp.float32),
        ],
    )
    def k(hs_hbm, row_hbm, col_hbm, agg_hbm, rbuf, cbuf, rows, zbuf, acc):
        cid = lax.axis_index("c")
        sid = lax.axis_index("s")

        pltpu.sync_copy(row_hbm.at[cid, sid], rbuf)
        pltpu.sync_copy(col_hbm.at[cid, sid], cbuf)
        _fill_const(zbuf, 64, 0.0)
        _zero_spmem(acc, zbuf, sid)
        plsc.subcore_barrier()

        @pl.loop(0, cpt)
        def _(j):
            pltpu.sync_copy(hs_hbm.at[rbuf.at[j]], rows)
            pltpu.sync_copy(rows, acc.at[cbuf.at[j]], add=True)

        plsc.subcore_barrier()
        base = sid * ROWS_PER_TILE
        pltpu.sync_copy(acc.at[pl.ds(base, ROWS_PER_TILE)],
                        agg_hbm.at[cid, pl.ds(base, ROWS_PER_TILE)])

    return k(hs, row_p, col_p)


def _mlp_call(x, W1, b1, W2, b2):
    def body(x_ref, w1_ref, b1_ref, w2_ref, b2_ref, o_ref):
        h = jnp.dot(x_ref[...], w1_ref[...],
                    preferred_element_type=jnp.float32) + b1_ref[...]
        h = jnp.maximum(h, 0.0)
        o_ref[...] = jnp.dot(h, w2_ref[...],
                             preferred_element_type=jnp.float32) + b2_ref[...]

    full = lambda i: (0, 0)
    return pl.pallas_call(
        body,
        grid=(N // RB,),
        in_specs=[
            pl.BlockSpec((RB, F), lambda i: (i, 0)),
            pl.BlockSpec((F, F), full),
            pl.BlockSpec((1, F), full),
            pl.BlockSpec((F, F), full),
            pl.BlockSpec((1, F), full),
        ],
        out_specs=pl.BlockSpec((RB, F), lambda i: (i, 0)),
        out_shape=jax.ShapeDtypeStruct((N, F), jnp.float32),
    )(x, W1, b1.reshape(1, F), W2, b2.reshape(1, F))


def _prep_call(h, deg16, gpad):
    def body(h_ref, d_ref, g_ref, hs_ref, out_ref):
        dis = lax.rsqrt(d_ref[:, :1] + 1.0)
        hv = h_ref[...]
        hs_ref[...] = dis * hv
        out_ref[...] = g_ref[0, 0] * hv

    return pl.pallas_call(
        body,
        grid=(N // RB,),
        in_specs=[
            pl.BlockSpec((RB, F), lambda i: (i, 0)),
            pl.BlockSpec((RB, 16), lambda i: (i, 0)),
            pl.BlockSpec((1, F), lambda i: (0, 0)),
        ],
        out_specs=[
            pl.BlockSpec((RB, F), lambda i: (i, 0)),
            pl.BlockSpec((RB, F), lambda i: (i, 0)),
        ],
        out_shape=[
            jax.ShapeDtypeStruct((N, F), jnp.float32),
            jax.ShapeDtypeStruct((N, F), jnp.float32),
        ],
    )(h, deg16, gpad)


def _dense_call(agg, hs, hidden, out, deg16, gpad, k):
    def body(a0_ref, a1_ref, hs_ref, hid_ref, out_ref, d_ref, g_ref,
             hidn_ref, outn_ref, hsn_ref):
        dis = lax.rsqrt(d_ref[:, :1] + 1.0)
        aggv = a0_ref[0] + a1_ref[0] + hs_ref[...]
        h_new = (1.0 - ALPHA) * (dis * aggv) + ALPHA * hid_ref[...]
        hidn_ref[...] = h_new
        outn_ref[...] = out_ref[...] + g_ref[0, k] * h_new
        hsn_ref[...] = dis * h_new

    blk = lambda i: (i, 0)
    return pl.pallas_call(
        body,
        grid=(N // RB,),
        in_specs=[
            pl.BlockSpec((1, RB, F), lambda i: (0, i, 0)),
            pl.BlockSpec((1, RB, F), lambda i: (1, i, 0)),
            pl.BlockSpec((RB, F), blk),
            pl.BlockSpec((RB, F), blk),
            pl.BlockSpec((RB, F), blk),
            pl.BlockSpec((RB, 16), blk),
            pl.BlockSpec((1, F), lambda i: (0, 0)),
        ],
        out_specs=[
            pl.BlockSpec((RB, F), blk),
            pl.BlockSpec((RB, F), blk),
            pl.BlockSpec((RB, F), blk),
        ],
        out_shape=[
            jax.ShapeDtypeStruct((N, F), jnp.float32),
            jax.ShapeDtypeStruct((N, F), jnp.float32),
            jax.ShapeDtypeStruct((N, F), jnp.float32),
        ],
    )(agg, agg, hs, hidden, out, deg16, gpad)


def _logsoftmax_call(out):
    def body(x_ref, o_ref):
        x = x_ref[...]
        m = jnp.max(x, axis=1, keepdims=True)
        s = x - m
        o_ref[...] = s - jnp.log(jnp.sum(jnp.exp(s), axis=1, keepdims=True))

    return pl.pallas_call(
        body,
        grid=(N // RB,),
        in_specs=[pl.BlockSpec((RB, F), lambda i: (i, 0))],
        out_specs=pl.BlockSpec((RB, F), lambda i: (i, 0)),
        out_shape=jax.ShapeDtypeStruct((N, F), jnp.float32),
    )(out)


def kernel(x, edge_index, W1, b1, W2, b2, gamma):
    E = edge_index.shape[1]
    row = edge_index[0].astype(jnp.int32)
    col = edge_index[1].astype(jnp.int32)

    ec = (E + NC - 1) // NC
    epc = NS * CH
    ec_pad = ((ec + epc - 1) // epc) * epc
    cpt = ec_pad // epc

    rows_p, cols_p = [], []
    for c in range(NC):
        lo = c * ec
        hi = min((c + 1) * ec, E)
        pad = ec_pad - (hi - lo)
        spread = jnp.arange(pad, dtype=jnp.int32) % 16
        rows_p.append(jnp.concatenate([row[lo:hi], spread]))
        cols_p.append(jnp.concatenate([col[lo:hi], N + spread]))
    row_p = jnp.stack(rows_p).reshape(NC, NS, cpt, CH)
    col_p = jnp.stack(cols_p).reshape(NC, NS, cpt, CH)

    gpad = jnp.zeros((1, F), jnp.float32).at[0, : K + 1].set(gamma)

    h = _mlp_call(x, W1, b1, W2, b2)
    deg16 = _deg_call(col_p, cpt)
    deg16 = deg16[:N]
    hs, out = _prep_call(h, deg16, gpad)
    hidden = h
    for k in range(1, K + 1):
        agg = _hop_call(hs, row_p, col_p, cpt)
        hidden, out, hs = _dense_call(agg, hs, hidden, out, deg16, gpad, k)
    return _logsoftmax_call(out)


# speedup vs baseline: 14.0194x; 14.0194x over previous
import functools

import jax
import jax.numpy as jnp
from jax import lax
from jax.experimental import pallas as pl
from jax.experimental.pallas import tpu as pltpu
from jax.experimental.pallas import tpu_sc as plsc

N = 10000
F = 128
K = 10
ALPHA = 0.1

NC = 2
NS = 16
CH = 128
NPAD = 10112
ROWS_PER_TILE = NPAD // NS

RB = 2000

_mesh = plsc.VectorSubcoreMesh(core_axis_name="c", subcore_axis_name="s")


def _fill_const(buf, nrows, val):
    @pl.loop(0, nrows)
    def _(i):
        @pl.loop(0, buf.shape[1], step=16)
        def _(j):
            buf[i, pl.ds(j, 16)] = jnp.full((16,), val, jnp.float32)


def _zero_spmem(acc, zbuf, tile_id):
    base = tile_id * ROWS_PER_TILE
    zrows = zbuf.shape[0]
    off = 0
    while off < ROWS_PER_TILE:
        sz = min(zrows, ROWS_PER_TILE - off)
        pltpu.sync_copy(zbuf.at[pl.ds(0, sz)], acc.at[pl.ds(base + off, sz)])
        off += sz


def _deg_call(col_p, cpt):
    @functools.partial(
        pl.kernel,
        out_type=jax.ShapeDtypeStruct((NPAD, 16), jnp.float32),
        mesh=_mesh,
        scratch_types=[
            pltpu.VMEM((cpt, CH), jnp.int32),
            pltpu.VMEM((CH, 16), jnp.float32),
            pltpu.VMEM((64, 16), jnp.float32),
            pltpu.VMEM_SHARED((NPAD, 16), jnp.float32),
        ],
    )
    def k(col_hbm, deg_hbm, cbuf, ones, zbuf, dacc):
        cid = lax.axis_index("c")
        sid = lax.axis_index("s")

        @pl.when(cid == 0)
        def _():
            _fill_const(ones, CH, 1.0)
            _fill_const(zbuf, 64, 0.0)
            base = sid * ROWS_PER_TILE
            off = 0
            while off < ROWS_PER_TILE:
                sz = min(64, ROWS_PER_TILE - off)
                pltpu.sync_copy(zbuf.at[pl.ds(0, sz)],
                                dacc.at[pl.ds(base + off, sz)])
                off += sz
            plsc.subcore_barrier()

            for seg in range(NC):
                pltpu.sync_copy(col_hbm.at[seg, sid], cbuf)

                @pl.loop(0, cpt)
                def _(j):
                    pltpu.sync_copy(ones, dacc.at[cbuf.at[j]], add=True)

            plsc.subcore_barrier()
            pltpu.sync_copy(dacc.at[pl.ds(base, ROWS_PER_TILE)],
                            deg_hbm.at[pl.ds(base, ROWS_PER_TILE)])

    return k(col_p)


def _hop_call(hs, row_p, col_p, cpt):
    @functools.partial(
        pl.kernel,
        out_type=jax.ShapeDtypeStruct((NC, NPAD, F), jnp.float32),
        mesh=_mesh,
        scratch_types=[
            pltpu.VMEM((cpt, CH), jnp.int32),
            pltpu.VMEM((cpt, CH), jnp.int32),
            pltpu.VMEM((CH, F), jnp.float32),
            pltpu.VMEM((64, F), jnp.float32),
            pltpu.VMEM_SHARED((NPAD, F), jnp.float32),
        ],
    )
    def k(hs_hbm, row_hbm, col_hbm, agg_hbm, rbuf, cbuf, rows, zbuf, acc):
        cid = lax.axis_index("c")
        sid = lax.axis_index("s")

        pltpu.sync_copy(row_hbm.at[cid, sid], rbuf)
        pltpu.sync_copy(col_hbm.at[cid, sid], cbuf)
        _fill_const(zbuf, 64, 0.0)
        _zero_spmem(acc, zbuf, sid)
        plsc.subcore_barrier()

        @pl.loop(0, cpt)
        def _(j):
            pltpu.sync_copy(hs_hbm.at[rbuf.at[j]], rows)
            pltpu.sync_copy(rows, acc.at[cbuf.at[j]], add=True)

        plsc.subcore_barrier()
        base = sid * ROWS_PER_TILE
        pltpu.sync_copy(acc.at[pl.ds(base, ROWS_PER_TILE)],
                        agg_hbm.at[cid, pl.ds(base, ROWS_PER_TILE)])

    return k(hs, row_p, col_p)


def _mlp_call(x, W1, b1, W2, b2):
    def body(x_ref, w1_ref, b1_ref, w2_ref, b2_ref, o_ref):
        h = jnp.dot(x_ref[...], w1_ref[...],
                    preferred_element_type=jnp.float32) + b1_ref[...]
        h = jnp.maximum(h, 0.0)
        o_ref[...] = jnp.dot(h, w2_ref[...],
                             preferred_element_type=jnp.float32) + b2_ref[...]

    full = lambda i: (0, 0)
    return pl.pallas_call(
        body,
        grid=(N // RB,),
        in_specs=[
            pl.BlockSpec((RB, F), lambda i: (i, 0)),
            pl.BlockSpec((F, F), full),
            pl.BlockSpec((1, F), full),
            pl.BlockSpec((F, F), full),
            pl.BlockSpec((1, F), full),
        ],
        out_specs=pl.BlockSpec((RB, F), lambda i: (i, 0)),
        out_shape=jax.ShapeDtypeStruct((N, F), jnp.float32),
    )(x, W1, b1.reshape(1, F), W2, b2.reshape(1, F))


def _prep_call(h, deg16, gpad):
    def body(h_ref, d_ref, g_ref, hs_ref, out_ref):
        dis = lax.rsqrt(d_ref[:, :1] + 1.0)
        hv = h_ref[...]
        hs_ref[...] = dis * hv
        out_ref[...] = g_ref[0, 0] * hv

    return pl.pallas_call(
        body,
        grid=(N // RB,),
        in_specs=[
            pl.BlockSpec((RB, F), lambda i: (i, 0)),
            pl.BlockSpec((RB, 16), lambda i: (i, 0)),
            pl.BlockSpec((1, F), lambda i: (0, 0)),
        ],
        out_specs=[
            pl.BlockSpec((RB, F), lambda i: (i, 0)),
            pl.BlockSpec((RB, F), lambda i: (i, 0)),
        ],
        out_shape=[
            jax.ShapeDtypeStruct((N, F), jnp.float32),
            jax.ShapeDtypeStruct((N, F), jnp.float32),
        ],
    )(h, deg16, gpad)


def _dense_call(agg, hs, hidden, out, deg16, gpad, k):
    def body(a0_ref, a1_ref, hs_ref, hid_ref, out_ref, d_ref, g_ref,
             hidn_ref, outn_ref, hsn_ref):
        dis = lax.rsqrt(d_ref[:, :1] + 1.0)
        aggv = a0_ref[0] + a1_ref[0] + hs_ref[...]
        h_new = (1.0 - ALPHA) * (dis * aggv) + ALPHA * hid_ref[...]
        hidn_ref[...] = h_new
        outn_ref[...] = out_ref[...] + g_ref[0, k] * h_new
        hsn_ref[...] = dis * h_new

    blk = lambda i: (i, 0)
    return pl.pallas_call(
        body,
        grid=(N // RB,),
        in_specs=[
            pl.BlockSpec((1, RB, F), lambda i: (0, i, 0)),
            pl.BlockSpec((1, RB, F), lambda i: (1, i, 0)),
            pl.BlockSpec((RB, F), blk),
            pl.BlockSpec((RB, F), blk),
            pl.BlockSpec((RB, F), blk),
            pl.BlockSpec((RB, 16), blk),
            pl.BlockSpec((1, F), lambda i: (0, 0)),
        ],
        out_specs=[
            pl.BlockSpec((RB, F), blk),
            pl.BlockSpec((RB, F), blk),
            pl.BlockSpec((RB, F), blk),
        ],
        out_shape=[
            jax.ShapeDtypeStruct((N, F), jnp.float32),
            jax.ShapeDtypeStruct((N, F), jnp.float32),
            jax.ShapeDtypeStruct((N, F), jnp.float32),
        ],
    )(agg, agg, hs, hidden, out, deg16, gpad)


def _logsoftmax_call(out):
    def body(x_ref, o_ref):
        x = x_ref[...]
        m = jnp.max(x, axis=1, keepdims=True)
        s = x - m
        o_ref[...] = s - jnp.log(jnp.sum(jnp.exp(s), axis=1, keepdims=True))

    return pl.pallas_call(
        body,
        grid=(N // RB,),
        in_specs=[pl.BlockSpec((RB, F), lambda i: (i, 0))],
        out_specs=pl.BlockSpec((RB, F), lambda i: (i, 0)),
        out_shape=jax.ShapeDtypeStruct((N, F), jnp.float32),
    )(out)


def kernel(x, edge_index, W1, b1, W2, b2, gamma):
    E = edge_index.shape[1]
    row = edge_index[0].astype(jnp.int32)
    col = edge_index[1].astype(jnp.int32)

    ec = (E + NC - 1) // NC
    epc = NS * CH
    ec_pad = ((ec + epc - 1) // epc) * epc
    cpt = ec_pad // epc

    rows_p, cols_p = [], []
    for c in range(NC):
        lo = c * ec
        hi = min((c + 1) * ec, E)
        pad = ec_pad - (hi - lo)
        spread = jnp.arange(pad, dtype=jnp.int32) % 16
        rows_p.append(jnp.concatenate([row[lo:hi], spread]))
        cols_p.append(jnp.concatenate([col[lo:hi], N + spread]))
    row_p = jnp.stack(rows_p).reshape(NC, NS, cpt, CH)
    col_p = jnp.stack(cols_p).reshape(NC, NS, cpt, CH)

    gpad = jnp.zeros((1, F), jnp.float32).at[0, : K + 1].set(gamma)

    h = _mlp_call(x, W1, b1, W2, b2)
    deg16 = _deg_call(col_p, cpt)
    deg16 = deg16[:N]
    hs, out = _prep_call(h, deg16, gpad)
    hidden = h
    for k in range(1, K + 1):
        agg = _hop_call(hs, row_p, col_p, cpt)
        hidden, out, hs = _dense_call(agg, hs, hidden, out, deg16, gpad, k)
    return _logsoftmax_call(out)


# speedup vs baseline: 21.8013x; 1.5551x over previous
import functools

import jax
import jax.numpy as jnp
from jax import lax
from jax.experimental import pallas as pl
from jax.experimental.pallas import tpu as pltpu
from jax.experimental.pallas import tpu_sc as plsc

N = 10000
F = 128
K = 10
ALPHA = 0.1

NC = 2
NS = 16
CH = 88
RING = 3
IDXPH = 2
NPAD = 10112
ROWS_PER_TILE = NPAD // NS

RB = 2000

_mesh = plsc.VectorSubcoreMesh(core_axis_name="c", subcore_axis_name="s")


def _fill_const(buf, nrows, val):
    @pl.loop(0, nrows)
    def _(i):
        @pl.loop(0, buf.shape[1], step=16)
        def _(j):
            buf[i, pl.ds(j, 16)] = jnp.full((16,), val, jnp.float32)


def _zero_spmem(acc, zbuf, tile_id):
    base = tile_id * ROWS_PER_TILE
    zrows = zbuf.shape[0]
    off = 0
    while off < ROWS_PER_TILE:
        sz = min(zrows, ROWS_PER_TILE - off)
        pltpu.sync_copy(zbuf.at[pl.ds(0, sz)], acc.at[pl.ds(base + off, sz)])
        off += sz


def _deg_call(col_p, cpp):
    @functools.partial(
        pl.kernel,
        out_type=jax.ShapeDtypeStruct((NPAD, 16), jnp.float32),
        mesh=_mesh,
        scratch_types=[
            pltpu.VMEM((cpp, CH), jnp.int32),
            pltpu.VMEM((CH, 16), jnp.float32),
            pltpu.VMEM((64, 16), jnp.float32),
            pltpu.VMEM_SHARED((NPAD, 16), jnp.float32),
        ],
    )
    def k(col_hbm, deg_hbm, cbuf, ones, zbuf, dacc):
        cid = lax.axis_index("c")
        sid = lax.axis_index("s")

        @pl.when(cid == 0)
        def _():
            _fill_const(ones, CH, 1.0)
            _fill_const(zbuf, 64, 0.0)
            base = sid * ROWS_PER_TILE
            off = 0
            while off < ROWS_PER_TILE:
                sz = min(64, ROWS_PER_TILE - off)
                pltpu.sync_copy(zbuf.at[pl.ds(0, sz)],
                                dacc.at[pl.ds(base + off, sz)])
                off += sz
            plsc.subcore_barrier()

            for seg in range(NC):
                for ph in range(IDXPH):
                    pltpu.sync_copy(col_hbm.at[seg, sid, ph], cbuf)

                    @pl.loop(0, cpp)
                    def _(j):
                        pltpu.sync_copy(ones, dacc.at[cbuf.at[j]], add=True)

            plsc.subcore_barrier()
            pltpu.sync_copy(dacc.at[pl.ds(base, ROWS_PER_TILE)],
                            deg_hbm.at[pl.ds(base, ROWS_PER_TILE)])

    return k(col_p)


def _hop_call(hs, row_p, col_p, cpt):
    cpp = cpt // IDXPH

    @functools.partial(
        pl.kernel,
        out_type=jax.ShapeDtypeStruct((NC, NPAD, F), jnp.float32),
        mesh=_mesh,
        scratch_types=[
            pltpu.VMEM((cpp, CH), jnp.int32),
            pltpu.VMEM((cpp, CH), jnp.int32),
            pltpu.VMEM((CH, F), jnp.float32),
            pltpu.VMEM((CH, F), jnp.float32),
            pltpu.VMEM((CH, F), jnp.float32),
            pltpu.VMEM_SHARED((NPAD, F), jnp.float32),
            pltpu.SemaphoreType.DMA,
            pltpu.SemaphoreType.DMA,
            pltpu.SemaphoreType.DMA,
            pltpu.SemaphoreType.DMA,
            pltpu.SemaphoreType.DMA,
            pltpu.SemaphoreType.DMA,
        ],
    )
    def k(hs_hbm, row_hbm, col_hbm, agg_hbm,
          rbuf, cbuf, d0, d1, d2, acc, g0, g1, g2, t0, t1, t2):
        cid = lax.axis_index("c")
        sid = lax.axis_index("s")
        bufs = (d0, d1, d2)
        gsem = (g0, g1, g2)
        ssem = (t0, t1, t2)

        _fill_const(d0, CH, 0.0)
        _zero_spmem(acc, d0, sid)
        plsc.subcore_barrier()

        for ph in range(IDXPH):
            pltpu.sync_copy(row_hbm.at[cid, sid, ph], rbuf)
            pltpu.sync_copy(col_hbm.at[cid, sid, ph], cbuf)

            for b in range(RING):
                pltpu.async_copy(hs_hbm.at[rbuf.at[b]], bufs[b], gsem[b])

            @pl.loop(0, cpp // RING)
            def _(q):
                j0 = q * RING
                for b in range(RING):
                    j = j0 + b
                    pltpu.make_async_copy(hs_hbm.at[rbuf.at[j]],
                                          bufs[b], gsem[b]).wait()
                    pltpu.async_copy(bufs[b], acc.at[cbuf.at[j]],
                                     ssem[b], add=True)
                    pb = (b + RING - 1) % RING
                    pj = j - 1

                    @pl.when(pj >= 0)
                    def _():
                        pltpu.make_async_copy(
                            bufs[pb], acc.at[cbuf.at[j]], ssem[pb]).wait()

                        @pl.when(pj + RING < cpp)
                        def _():
                            pltpu.async_copy(hs_hbm.at[rbuf.at[pj + RING]],
                                             bufs[pb], gsem[pb])

            pltpu.make_async_copy(bufs[RING - 1], acc.at[cbuf.at[0]],
                                  ssem[RING - 1]).wait()

        plsc.subcore_barrier()
        base = sid * ROWS_PER_TILE
        pltpu.sync_copy(acc.at[pl.ds(base, ROWS_PER_TILE)],
                        agg_hbm.at[cid, pl.ds(base, ROWS_PER_TILE)])

    return k(hs, row_p, col_p)


def _mlp_call(x, W1, b1, W2, b2):
    def body(x_ref, w1_ref, b1_ref, w2_ref, b2_ref, o_ref):
        h = jnp.dot(x_ref[...], w1_ref[...],
                    preferred_element_type=jnp.float32) + b1_ref[...]
        h = jnp.maximum(h, 0.0)
        o_ref[...] = jnp.dot(h, w2_ref[...],
                             preferred_element_type=jnp.float32) + b2_ref[...]

    full = lambda i: (0, 0)
    return pl.pallas_call(
        body,
        grid=(N // RB,),
        in_specs=[
            pl.BlockSpec((RB, F), lambda i: (i, 0)),
            pl.BlockSpec((F, F), full),
            pl.BlockSpec((1, F), full),
            pl.BlockSpec((F, F), full),
            pl.BlockSpec((1, F), full),
        ],
        out_specs=pl.BlockSpec((RB, F), lambda i: (i, 0)),
        out_shape=jax.ShapeDtypeStruct((N, F), jnp.float32),
    )(x, W1, b1.reshape(1, F), W2, b2.reshape(1, F))


def _prep_call(h, deg16, gpad):
    def body(h_ref, d_ref, g_ref, hs_ref, out_ref):
        dis = lax.rsqrt(d_ref[:, :1] + 1.0)
        hv = h_ref[...]
        hs_ref[...] = dis * hv
        out_ref[...] = g_ref[0, 0] * hv

    return pl.pallas_call(
        body,
        grid=(N // RB,),
        in_specs=[
            pl.BlockSpec((RB, F), lambda i: (i, 0)),
            pl.BlockSpec((RB, 16), lambda i: (i, 0)),
            pl.BlockSpec((1, F), lambda i: (0, 0)),
        ],
        out_specs=[
            pl.BlockSpec((RB, F), lambda i: (i, 0)),
            pl.BlockSpec((RB, F), lambda i: (i, 0)),
        ],
        out_shape=[
            jax.ShapeDtypeStruct((N, F), jnp.float32),
            jax.ShapeDtypeStruct((N, F), jnp.float32),
        ],
    )(h, deg16, gpad)


def _dense_call(agg, hs, hidden, out, deg16, gpad, k):
    def body(a0_ref, a1_ref, hs_ref, hid_ref, out_ref, d_ref, g_ref,
             hidn_ref, outn_ref, hsn_ref):
        dis = lax.rsqrt(d_ref[:, :1] + 1.0)
        aggv = a0_ref[0] + a1_ref[0] + hs_ref[...]
        h_new = (1.0 - ALPHA) * (dis * aggv) + ALPHA * hid_ref[...]
        hidn_ref[...] = h_new
        outn_ref[...] = out_ref[...] + g_ref[0, k] * h_new
        hsn_ref[...] = dis * h_new

    blk = lambda i: (i, 0)
    return pl.pallas_call(
        body,
        grid=(N // RB,),
        in_specs=[
            pl.BlockSpec((1, RB, F), lambda i: (0, i, 0)),
            pl.BlockSpec((1, RB, F), lambda i: (1, i, 0)),
            pl.BlockSpec((RB, F), blk),
            pl.BlockSpec((RB, F), blk),
            pl.BlockSpec((RB, F), blk),
            pl.BlockSpec((RB, 16), blk),
            pl.BlockSpec((1, F), lambda i: (0, 0)),
        ],
        out_specs=[
            pl.BlockSpec((RB, F), blk),
            pl.BlockSpec((RB, F), blk),
            pl.BlockSpec((RB, F), blk),
        ],
        out_shape=[
            jax.ShapeDtypeStruct((N, F), jnp.float32),
            jax.ShapeDtypeStruct((N, F), jnp.float32),
            jax.ShapeDtypeStruct((N, F), jnp.float32),
        ],
    )(agg, agg, hs, hidden, out, deg16, gpad)


def _logsoftmax_call(out):
    def body(x_ref, o_ref):
        x = x_ref[...]
        m = jnp.max(x, axis=1, keepdims=True)
        s = x - m
        o_ref[...] = s - jnp.log(jnp.sum(jnp.exp(s), axis=1, keepdims=True))

    return pl.pallas_call(
        body,
        grid=(N // RB,),
        in_specs=[pl.BlockSpec((RB, F), lambda i: (i, 0))],
        out_specs=pl.BlockSpec((RB, F), lambda i: (i, 0)),
        out_shape=jax.ShapeDtypeStruct((N, F), jnp.float32),
    )(out)


def kernel(x, edge_index, W1, b1, W2, b2, gamma):
    E = edge_index.shape[1]
    row = edge_index[0].astype(jnp.int32)
    col = edge_index[1].astype(jnp.int32)

    ec = (E + NC - 1) // NC
    epc = NS * CH * RING * IDXPH
    ec_pad = ((ec + epc - 1) // epc) * epc
    cpt = ec_pad // (NS * CH)

    rows_p, cols_p = [], []
    for c in range(NC):
        lo = c * ec
        hi = min((c + 1) * ec, E)
        pad = ec_pad - (hi - lo)
        spread = jnp.arange(pad, dtype=jnp.int32) % 16
        rows_p.append(jnp.concatenate([row[lo:hi], spread]))
        cols_p.append(jnp.concatenate([col[lo:hi], N + spread]))
    cpp = cpt // IDXPH
    row_p = jnp.stack(rows_p).reshape(NC, NS, IDXPH, cpp, CH)
    col_p = jnp.stack(cols_p).reshape(NC, NS, IDXPH, cpp, CH)

    gpad = jnp.zeros((1, F), jnp.float32).at[0, : K + 1].set(gamma)

    h = _mlp_call(x, W1, b1, W2, b2)
    deg16 = _deg_call(col_p, cpp)
    deg16 = deg16[:N]
    hs, out = _prep_call(h, deg16, gpad)
    hidden = h
    for k in range(1, K + 1):
        agg = _hop_call(hs, row_p, col_p, cpt)
        hidden, out, hs = _dense_call(agg, hs, hidden, out, deg16, gpad, k)
    return _logsoftmax_call(out)
